# baseline (speedup 1.0000x reference)
import functools

import jax
import jax.numpy as jnp
from jax import lax
from jax.experimental import pallas as pl
from jax.experimental.pallas import tpu as pltpu
from jax.experimental.pallas import tpu_sc as plsc

N_NODES = 10000
D = 128
NEG_SLOPE = 0.2

NC, NS, L = 2, 16, 16
NW = NC * NS
N_PAD = 10240
E_REAL = N_NODES + 320000
CHUNK = 128
N_CHUNKS = 81
EPT = CHUNK * N_CHUNKS
E_PAD = NW * EPT
ROWS_2D = NW * N_CHUNKS
STRIPE = N_PAD // NS
C0, C1 = 89, 73

_mesh = plsc.VectorSubcoreMesh(core_axis_name="c", subcore_axis_name="s")
_sc_params = pltpu.CompilerParams(use_tc_tiling_on_sc=False,
                                  needs_layout_passes=False)


def _pre_body(x_ref, w_ref, asv_ref, adv_ref, h_ref, as_ref, ad_ref):
    h = jnp.dot(x_ref[...], w_ref[...], preferred_element_type=jnp.float32)
    h_ref[...] = h
    as_ref[...] = jnp.dot(h, asv_ref[...], preferred_element_type=jnp.float32)
    ad_ref[...] = jnp.dot(h, adv_ref[...], preferred_element_type=jnp.float32)


def _pre(x, w, as_vec, ad_vec):
    blk = 1000
    grid = N_NODES // blk
    return pl.pallas_call(
        _pre_body,
        grid=(grid,),
        in_specs=[
            pl.BlockSpec((blk, D), lambda i: (i, 0)),
            pl.BlockSpec((D, D), lambda i: (0, 0)),
            pl.BlockSpec((D, 1), lambda i: (0, 0)),
            pl.BlockSpec((D, 1), lambda i: (0, 0)),
        ],
        out_specs=[
            pl.BlockSpec((blk, D), lambda i: (i, 0)),
            pl.BlockSpec((blk, 1), lambda i: (i, 0)),
            pl.BlockSpec((blk, 1), lambda i: (i, 0)),
        ],
        out_shape=[
            jax.ShapeDtypeStruct((N_PAD, D), jnp.float32),
            jax.ShapeDtypeStruct((N_PAD, 1), jnp.float32),
            jax.ShapeDtypeStruct((N_PAD, 1), jnp.float32),
        ],
    )(x, w, as_vec, ad_vec)


@functools.partial(
    pl.kernel,
    out_type=[
        jax.ShapeDtypeStruct((NC, N_PAD), jnp.float32),
        jax.ShapeDtypeStruct((NC, N_PAD, D), jnp.float32),
    ],
    mesh=_mesh,
    scratch_types=[
        pltpu.VMEM((2, 2, CHUNK), jnp.int32),
        pltpu.VMEM((2, CHUNK), jnp.float32),
        pltpu.VMEM((2, CHUNK), jnp.float32),
        pltpu.VMEM((2, CHUNK), jnp.float32),
        pltpu.VMEM((STRIPE,), jnp.float32),
        pltpu.VMEM((2, CHUNK, D), jnp.float32),
        pltpu.VMEM_SHARED((N_PAD, D), jnp.float32),
        pltpu.VMEM_SHARED((N_PAD,), jnp.float32),
        pltpu.SemaphoreType.DMA,
        pltpu.SemaphoreType.DMA,
        pltpu.SemaphoreType.DMA,
        pltpu.SemaphoreType.DMA,
    ],
    compiler_params=_sc_params,
)
def _edge_pass(h_hbm, asrc_hbm, adst_hbm, ei_hbm, denom_hbm, out_hbm,
               idx_v, asb_v, adb_v, wv_v, dstage_v, rowb_v, out_sh, denom_sh,
               gsem, lsem, ssem, dsem):
    cid = lax.axis_index("c")
    sid = lax.axis_index("s")
    nch = jnp.where(cid == 0, C0, C1)
    base = cid * NS * C0 + sid * nch

    def zrow(j, _):
        for v in range(D // L):
            rowb_v[0, j, pl.ds(v * L, L)] = jnp.zeros((L,), jnp.float32)
        return _
    lax.fori_loop(0, CHUNK, zrow, None)
    for t in range(STRIPE // CHUNK):
        pltpu.sync_copy(rowb_v.at[0],
                        out_sh.at[pl.ds(sid * STRIPE + t * CHUNK, CHUNK)])
    def zden(k, _):
        dstage_v[pl.ds(k * L, L)] = jnp.zeros((L,), jnp.float32)
        return _
    lax.fori_loop(0, STRIPE // L, zden, None)
    pltpu.sync_copy(dstage_v, denom_sh.at[pl.ds(sid * STRIPE, STRIPE)])
    plsc.subcore_barrier()

    def fetch(c, slot):
        row = base + c
        pltpu.sync_copy(ei_hbm.at[row], idx_v.at[slot])
        pltpu.async_copy(asrc_hbm.at[idx_v.at[slot, 0]], asb_v.at[slot], lsem)
        pltpu.async_copy(adst_hbm.at[idx_v.at[slot, 1]], adb_v.at[slot], lsem)
        pltpu.async_copy(h_hbm.at[idx_v.at[slot, 0]], rowb_v.at[slot], gsem)

    fetch(0, 0)

    def chunk_body(c, _):
        slot = lax.rem(c, 2)
        nslot = lax.rem(c + 1, 2)

        @pl.when(c + 1 < nch)
        def _prefetch():
            @pl.when(c >= 1)
            def _drain_scatters():
                pltpu.make_async_copy(
                    rowb_v.at[nslot], out_sh.at[idx_v.at[nslot, 1]], ssem).wait()
                pltpu.make_async_copy(
                    wv_v.at[nslot], denom_sh.at[idx_v.at[nslot, 1]], dsem).wait()
            fetch(c + 1, nslot)

        pltpu.make_async_copy(
            asrc_hbm.at[idx_v.at[slot, 0]], asb_v.at[slot], lsem).wait()
        pltpu.make_async_copy(
            adst_hbm.at[idx_v.at[slot, 1]], adb_v.at[slot], lsem).wait()
        pltpu.make_async_copy(
            h_hbm.at[idx_v.at[slot, 0]], rowb_v.at[slot], gsem).wait()

        for k in range(CHUNK // L):
            e = asb_v[slot, pl.ds(k * L, L)] + adb_v[slot, pl.ds(k * L, L)]
            e = jnp.where(e >= 0.0, e, e * NEG_SLOPE)
            w = jnp.exp(e)
            gidx = (base + c) * CHUNK + k * L + lax.iota(jnp.int32, L)
            wv_v[slot, pl.ds(k * L, L)] = jnp.where(gidx < E_REAL, w, 0.0)

        pltpu.async_copy(wv_v.at[slot], denom_sh.at[idx_v.at[slot, 1]], dsem,
                         add=True)

        @plsc.parallel_loop(0, CHUNK, step=1, unroll=4)
        def scale(j):
            a = plsc.load_gather(
                wv_v, [jnp.full((L,), slot, jnp.int32),
                       jnp.full((L,), j, jnp.int32)])
            for v in range(D // L):
                rowb_v[slot, j, pl.ds(v * L, L)] = (
                    rowb_v[slot, j, pl.ds(v * L, L)] * a)

        pltpu.async_copy(rowb_v.at[slot], out_sh.at[idx_v.at[slot, 1]], ssem,
                         add=True)
        return _
    lax.fori_loop(0, nch, chunk_body, None)

    for s in range(2):
        pltpu.make_async_copy(
            rowb_v.at[s], out_sh.at[idx_v.at[s, 1]], ssem).wait()
        pltpu.make_async_copy(
            wv_v.at[s], denom_sh.at[idx_v.at[s, 1]], dsem).wait()

    plsc.subcore_barrier()
    pltpu.sync_copy(denom_sh.at[pl.ds(sid * STRIPE, STRIPE)], dstage_v)
    pltpu.sync_copy(dstage_v, denom_hbm.at[cid].at[pl.ds(sid * STRIPE, STRIPE)])
    for t in range(STRIPE // CHUNK):
        base = sid * STRIPE + t * CHUNK
        pltpu.sync_copy(out_sh.at[pl.ds(base, CHUNK)], rowb_v.at[0])
        pltpu.sync_copy(rowb_v.at[0], out_hbm.at[cid].at[pl.ds(base, CHUNK)])


@functools.partial(
    pl.kernel,
    out_type=jax.ShapeDtypeStruct((ROWS_2D, CHUNK), jnp.float32),
    mesh=_mesh,
    scratch_types=[
        pltpu.VMEM((N_PAD,), jnp.float32),
        pltpu.VMEM((N_PAD,), jnp.float32),
        pltpu.VMEM((N_PAD,), jnp.float32),
        pltpu.VMEM((N_PAD,), jnp.float32),
        pltpu.VMEM((N_CHUNKS, 2, CHUNK), jnp.int32),
        pltpu.VMEM((N_CHUNKS, CHUNK), jnp.float32),
    ],
    compiler_params=_sc_params,
)
def _alpha_pass(asrc_hbm, adst_hbm, ei_hbm, denom_hbm, alpha_hbm,
                asrc_v, adst_v, den_v, den1_v, ei_v, al_v):
    cid = lax.axis_index("c")
    sid = lax.axis_index("s")
    wid = sid * NC + cid

    pltpu.sync_copy(asrc_hbm, asrc_v)
    pltpu.sync_copy(adst_hbm, adst_v)
    pltpu.sync_copy(denom_hbm.at[0], den_v)
    pltpu.sync_copy(denom_hbm.at[1], den1_v)
    pltpu.sync_copy(ei_hbm.at[pl.ds(wid * N_CHUNKS, N_CHUNKS)], ei_v)
    def dsum(k, _):
        den_v[pl.ds(k * L, L)] = den_v[pl.ds(k * L, L)] + den1_v[pl.ds(k * L, L)]
        return _
    lax.fori_loop(0, N_PAD // L, dsum, None)

    def chunk_body(c, _):
        for k in range(CHUNK // L):
            s_idx = ei_v[c, 0, pl.ds(k * L, L)]
            d_idx = ei_v[c, 1, pl.ds(k * L, L)]
            e = plsc.load_gather(asrc_v, [s_idx]) + plsc.load_gather(adst_v, [d_idx])
            e = jnp.where(e >= 0.0, e, e * NEG_SLOPE)
            w = jnp.exp(e)
            gidx = wid * EPT + c * CHUNK + k * L + lax.iota(jnp.int32, L)
            w = jnp.where(gidx < E_REAL, w, 0.0)
            den = plsc.load_gather(den_v, [d_idx])
            al_v[c, pl.ds(k * L, L)] = w / (den + 1e-16)
        return _
    lax.fori_loop(0, N_CHUNKS, chunk_body, None)
    pltpu.sync_copy(al_v, alpha_hbm.at[pl.ds(wid * N_CHUNKS, N_CHUNKS)])


def _post_body(op_ref, den_ref, bias_ref, g_ref, b_ref, out_ref):
    d = den_ref[0] + den_ref[1] + 1e-16
    o = (op_ref[0] + op_ref[1]) / d + bias_ref[...]
    mu = jnp.mean(o, axis=-1, keepdims=True)
    var = jnp.mean((o - mu) * (o - mu), axis=-1, keepdims=True)
    out_ref[...] = (o - mu) * lax.rsqrt(var + 1e-5) * g_ref[...] + b_ref[...]


def _post(out_partials, denom_partials, bias, gamma, beta):
    blk = 1000
    grid = N_NODES // blk
    return pl.pallas_call(
        _post_body,
        grid=(grid,),
        in_specs=[
            pl.BlockSpec((NC, blk, D), lambda i: (0, i, 0)),
            pl.BlockSpec((NC, blk, 1), lambda i: (0, i, 0)),
            pl.BlockSpec((1, D), lambda i: (0, 0)),
            pl.BlockSpec((1, D), lambda i: (0, 0)),
            pl.BlockSpec((1, D), lambda i: (0, 0)),
        ],
        out_specs=pl.BlockSpec((blk, D), lambda i: (i, 0)),
        out_shape=jax.ShapeDtypeStruct((N_NODES, D), jnp.float32),
    )(out_partials, denom_partials, bias, gamma, beta)


def kernel(X, edge_index, edge_attr, W, att_src, att_dst, bias, ln_gamma, ln_beta):
    N = X.shape[0]
    loops = jnp.arange(N, dtype=edge_index.dtype)
    ei = jnp.concatenate([edge_index, jnp.stack([loops, loops], axis=0)], axis=1)

    pad = E_PAD - E_REAL
    src = jnp.concatenate([ei[0], jnp.zeros((pad,), ei.dtype)]).astype(jnp.int32)
    dst = jnp.concatenate([ei[1], jnp.zeros((pad,), ei.dtype)]).astype(jnp.int32)
    src2d = src.reshape(ROWS_2D, CHUNK)
    dst2d = dst.reshape(ROWS_2D, CHUNK)
    ei3 = jnp.stack([src2d, dst2d], axis=1)

    as_vec = att_src.reshape(D, 1)
    ad_vec = att_dst.reshape(D, 1)

    h, a_s, a_d = _pre(X, W, as_vec, ad_vec)
    a_s = a_s.reshape(N_PAD)
    a_d = a_d.reshape(N_PAD)

    denom_p, out_p = _edge_pass(h, a_s, a_d, ei3)
    alpha2d = _alpha_pass(a_s, a_d, ei3, denom_p)
    h_norm = _post(out_p, denom_p.reshape(NC, N_PAD, 1), bias.reshape(1, D),
                   ln_gamma.reshape(1, D), ln_beta.reshape(1, D))

    alpha = alpha2d.reshape(E_PAD)[:E_REAL].reshape(E_REAL, 1)
    return (h_norm, edge_index, edge_attr, ei, alpha)


# speedup vs baseline: 1.0541x; 1.0541x over previous
import functools

import jax
import jax.numpy as jnp
from jax import lax
from jax.experimental import pallas as pl
from jax.experimental.pallas import tpu as pltpu
from jax.experimental.pallas import tpu_sc as plsc

N_NODES = 10000
D = 128
NEG_SLOPE = 0.2

NC, NS, L = 2, 16, 16
NW = NC * NS
N_PAD = 10240
E_REAL = N_NODES + 320000
CHUNK = 128
N_CHUNKS = 81
EPT = CHUNK * N_CHUNKS
E_PAD = NW * EPT
ROWS_2D = NW * N_CHUNKS
STRIPE = N_PAD // NS
C0, C1 = 100, 62

_mesh = plsc.VectorSubcoreMesh(core_axis_name="c", subcore_axis_name="s")
_sc_params = pltpu.CompilerParams(use_tc_tiling_on_sc=False,
                                  needs_layout_passes=False)


def _pre_body(x_ref, w_ref, asv_ref, adv_ref, h_ref, as_ref, ad_ref):
    h = jnp.dot(x_ref[...], w_ref[...], preferred_element_type=jnp.float32)
    h_ref[...] = h
    as_ref[...] = jnp.dot(h, asv_ref[...], preferred_element_type=jnp.float32)
    ad_ref[...] = jnp.dot(h, adv_ref[...], preferred_element_type=jnp.float32)


def _pre(x, w, as_vec, ad_vec):
    blk = 1000
    grid = N_NODES // blk
    return pl.pallas_call(
        _pre_body,
        grid=(grid,),
        in_specs=[
            pl.BlockSpec((blk, D), lambda i: (i, 0)),
            pl.BlockSpec((D, D), lambda i: (0, 0)),
            pl.BlockSpec((D, 1), lambda i: (0, 0)),
            pl.BlockSpec((D, 1), lambda i: (0, 0)),
        ],
        out_specs=[
            pl.BlockSpec((blk, D), lambda i: (i, 0)),
            pl.BlockSpec((blk, 1), lambda i: (i, 0)),
            pl.BlockSpec((blk, 1), lambda i: (i, 0)),
        ],
        out_shape=[
            jax.ShapeDtypeStruct((N_PAD, D), jnp.float32),
            jax.ShapeDtypeStruct((N_PAD, 1), jnp.float32),
            jax.ShapeDtypeStruct((N_PAD, 1), jnp.float32),
        ],
    )(x, w, as_vec, ad_vec)


@functools.partial(
    pl.kernel,
    out_type=[
        jax.ShapeDtypeStruct((NC, N_PAD), jnp.float32),
        jax.ShapeDtypeStruct((NC, N_PAD, D), jnp.float32),
    ],
    mesh=_mesh,
    scratch_types=[
        pltpu.VMEM((2, 2, CHUNK), jnp.int32),
        pltpu.VMEM((2, CHUNK), jnp.float32),
        pltpu.VMEM((2, CHUNK), jnp.float32),
        pltpu.VMEM((2, CHUNK), jnp.float32),
        pltpu.VMEM((STRIPE,), jnp.float32),
        pltpu.VMEM((2, CHUNK, D), jnp.float32),
        pltpu.VMEM_SHARED((N_PAD, D), jnp.float32),
        pltpu.VMEM_SHARED((N_PAD,), jnp.float32),
        pltpu.SemaphoreType.DMA,
        pltpu.SemaphoreType.DMA,
        pltpu.SemaphoreType.DMA,
        pltpu.SemaphoreType.DMA,
    ],
    compiler_params=_sc_params,
)
def _edge_pass(h_hbm, asrc_hbm, adst_hbm, ei_hbm, denom_hbm, out_hbm,
               idx_v, asb_v, adb_v, wv_v, dstage_v, rowb_v, out_sh, denom_sh,
               gsem, lsem, ssem, dsem):
    cid = lax.axis_index("c")
    sid = lax.axis_index("s")
    nch = jnp.where(cid == 0, C0, C1)
    base = cid * NS * C0 + sid * nch

    def zrow(j, _):
        for v in range(D // L):
            rowb_v[0, j, pl.ds(v * L, L)] = jnp.zeros((L,), jnp.float32)
        return _
    lax.fori_loop(0, CHUNK, zrow, None)
    for t in range(STRIPE // CHUNK):
        pltpu.sync_copy(rowb_v.at[0],
                        out_sh.at[pl.ds(sid * STRIPE + t * CHUNK, CHUNK)])
    def zden(k, _):
        dstage_v[pl.ds(k * L, L)] = jnp.zeros((L,), jnp.float32)
        return _
    lax.fori_loop(0, STRIPE // L, zden, None)
    pltpu.sync_copy(dstage_v, denom_sh.at[pl.ds(sid * STRIPE, STRIPE)])
    plsc.subcore_barrier()

    def fetch(c, slot):
        row = base + c
        pltpu.sync_copy(ei_hbm.at[row], idx_v.at[slot])
        pltpu.async_copy(asrc_hbm.at[idx_v.at[slot, 0]], asb_v.at[slot], lsem)
        pltpu.async_copy(adst_hbm.at[idx_v.at[slot, 1]], adb_v.at[slot], lsem)
        pltpu.async_copy(h_hbm.at[idx_v.at[slot, 0]], rowb_v.at[slot], gsem)

    fetch(0, 0)

    def chunk_body(c, _):
        slot = lax.rem(c, 2)
        nslot = lax.rem(c + 1, 2)

        @pl.when(c + 1 < nch)
        def _prefetch():
            @pl.when(c >= 1)
            def _drain_scatters():
                pltpu.make_async_copy(
                    rowb_v.at[nslot], out_sh.at[idx_v.at[nslot, 1]], ssem).wait()
                pltpu.make_async_copy(
                    wv_v.at[nslot], denom_sh.at[idx_v.at[nslot, 1]], dsem).wait()
            fetch(c + 1, nslot)

        pltpu.make_async_copy(
            asrc_hbm.at[idx_v.at[slot, 0]], asb_v.at[slot], lsem).wait()
        pltpu.make_async_copy(
            adst_hbm.at[idx_v.at[slot, 1]], adb_v.at[slot], lsem).wait()
        pltpu.make_async_copy(
            h_hbm.at[idx_v.at[slot, 0]], rowb_v.at[slot], gsem).wait()

        for k in range(CHUNK // L):
            e = asb_v[slot, pl.ds(k * L, L)] + adb_v[slot, pl.ds(k * L, L)]
            e = jnp.where(e >= 0.0, e, e * NEG_SLOPE)
            w = jnp.exp(e)
            gidx = (base + c) * CHUNK + k * L + lax.iota(jnp.int32, L)
            wv_v[slot, pl.ds(k * L, L)] = jnp.where(gidx < E_REAL, w, 0.0)

        pltpu.async_copy(wv_v.at[slot], denom_sh.at[idx_v.at[slot, 1]], dsem,
                         add=True)

        @plsc.parallel_loop(0, CHUNK, step=1, unroll=4)
        def scale(j):
            a = plsc.load_gather(
                wv_v, [jnp.full((L,), slot, jnp.int32),
                       jnp.full((L,), j, jnp.int32)])
            for v in range(D // L):
                rowb_v[slot, j, pl.ds(v * L, L)] = (
                    rowb_v[slot, j, pl.ds(v * L, L)] * a)

        pltpu.async_copy(rowb_v.at[slot], out_sh.at[idx_v.at[slot, 1]], ssem,
                         add=True)
        return _
    lax.fori_loop(0, nch, chunk_body, None)

    for s in range(2):
        pltpu.make_async_copy(
            rowb_v.at[s], out_sh.at[idx_v.at[s, 1]], ssem).wait()
        pltpu.make_async_copy(
            wv_v.at[s], denom_sh.at[idx_v.at[s, 1]], dsem).wait()

    plsc.subcore_barrier()
    pltpu.sync_copy(denom_sh.at[pl.ds(sid * STRIPE, STRIPE)], dstage_v)
    pltpu.sync_copy(dstage_v, denom_hbm.at[cid].at[pl.ds(sid * STRIPE, STRIPE)])
    for t in range(STRIPE // CHUNK):
        base = sid * STRIPE + t * CHUNK
        pltpu.sync_copy(out_sh.at[pl.ds(base, CHUNK)], rowb_v.at[0])
        pltpu.sync_copy(rowb_v.at[0], out_hbm.at[cid].at[pl.ds(base, CHUNK)])


@functools.partial(
    pl.kernel,
    out_type=jax.ShapeDtypeStruct((ROWS_2D, CHUNK), jnp.float32),
    mesh=_mesh,
    scratch_types=[
        pltpu.VMEM((N_PAD,), jnp.float32),
        pltpu.VMEM((N_PAD,), jnp.float32),
        pltpu.VMEM((N_PAD,), jnp.float32),
        pltpu.VMEM((N_PAD,), jnp.float32),
        pltpu.VMEM((N_CHUNKS, 2, CHUNK), jnp.int32),
        pltpu.VMEM((N_CHUNKS, CHUNK), jnp.float32),
    ],
    compiler_params=_sc_params,
)
def _alpha_pass(asrc_hbm, adst_hbm, ei_hbm, denom_hbm, alpha_hbm,
                asrc_v, adst_v, den_v, den1_v, ei_v, al_v):
    cid = lax.axis_index("c")
    sid = lax.axis_index("s")
    wid = sid * NC + cid

    pltpu.sync_copy(asrc_hbm, asrc_v)
    pltpu.sync_copy(adst_hbm, adst_v)
    pltpu.sync_copy(denom_hbm.at[0], den_v)
    pltpu.sync_copy(denom_hbm.at[1], den1_v)
    pltpu.sync_copy(ei_hbm.at[pl.ds(wid * N_CHUNKS, N_CHUNKS)], ei_v)
    def dsum(k, _):
        den_v[pl.ds(k * L, L)] = den_v[pl.ds(k * L, L)] + den1_v[pl.ds(k * L, L)]
        return _
    lax.fori_loop(0, N_PAD // L, dsum, None)

    def chunk_body(c, _):
        for k in range(CHUNK // L):
            s_idx = ei_v[c, 0, pl.ds(k * L, L)]
            d_idx = ei_v[c, 1, pl.ds(k * L, L)]
            e = plsc.load_gather(asrc_v, [s_idx]) + plsc.load_gather(adst_v, [d_idx])
            e = jnp.where(e >= 0.0, e, e * NEG_SLOPE)
            w = jnp.exp(e)
            gidx = wid * EPT + c * CHUNK + k * L + lax.iota(jnp.int32, L)
            w = jnp.where(gidx < E_REAL, w, 0.0)
            den = plsc.load_gather(den_v, [d_idx])
            al_v[c, pl.ds(k * L, L)] = w / (den + 1e-16)
        return _
    lax.fori_loop(0, N_CHUNKS, chunk_body, None)
    pltpu.sync_copy(al_v, alpha_hbm.at[pl.ds(wid * N_CHUNKS, N_CHUNKS)])


def _post_body(op_ref, den_ref, bias_ref, g_ref, b_ref, out_ref):
    d = den_ref[0] + den_ref[1] + 1e-16
    o = (op_ref[0] + op_ref[1]) / d + bias_ref[...]
    mu = jnp.mean(o, axis=-1, keepdims=True)
    var = jnp.mean((o - mu) * (o - mu), axis=-1, keepdims=True)
    out_ref[...] = (o - mu) * lax.rsqrt(var + 1e-5) * g_ref[...] + b_ref[...]


def _post(out_partials, denom_partials, bias, gamma, beta):
    blk = 1000
    grid = N_NODES // blk
    return pl.pallas_call(
        _post_body,
        grid=(grid,),
        in_specs=[
            pl.BlockSpec((NC, blk, D), lambda i: (0, i, 0)),
            pl.BlockSpec((NC, blk, 1), lambda i: (0, i, 0)),
            pl.BlockSpec((1, D), lambda i: (0, 0)),
            pl.BlockSpec((1, D), lambda i: (0, 0)),
            pl.BlockSpec((1, D), lambda i: (0, 0)),
        ],
        out_specs=pl.BlockSpec((blk, D), lambda i: (i, 0)),
        out_shape=jax.ShapeDtypeStruct((N_NODES, D), jnp.float32),
    )(out_partials, denom_partials, bias, gamma, beta)


def kernel(X, edge_index, edge_attr, W, att_src, att_dst, bias, ln_gamma, ln_beta):
    N = X.shape[0]
    loops = jnp.arange(N, dtype=edge_index.dtype)
    ei = jnp.concatenate([edge_index, jnp.stack([loops, loops], axis=0)], axis=1)

    pad = E_PAD - E_REAL
    src = jnp.concatenate([ei[0], jnp.zeros((pad,), ei.dtype)]).astype(jnp.int32)
    dst = jnp.concatenate([ei[1], jnp.zeros((pad,), ei.dtype)]).astype(jnp.int32)
    src2d = src.reshape(ROWS_2D, CHUNK)
    dst2d = dst.reshape(ROWS_2D, CHUNK)
    ei3 = jnp.stack([src2d, dst2d], axis=1)

    as_vec = att_src.reshape(D, 1)
    ad_vec = att_dst.reshape(D, 1)

    h, a_s, a_d = _pre(X, W, as_vec, ad_vec)
    a_s = a_s.reshape(N_PAD)
    a_d = a_d.reshape(N_PAD)

    denom_p, out_p = _edge_pass(h, a_s, a_d, ei3)
    alpha2d = _alpha_pass(a_s, a_d, ei3, denom_p)
    h_norm = _post(out_p, denom_p.reshape(NC, N_PAD, 1), bias.reshape(1, D),
                   ln_gamma.reshape(1, D), ln_beta.reshape(1, D))

    alpha = alpha2d.reshape(E_PAD)[:E_REAL].reshape(E_REAL, 1)
    return (h_norm, edge_index, edge_attr, ei, alpha)
